# baseline (speedup 1.0000x reference)
import functools

import jax
import jax.numpy as jnp
from jax import lax
from jax.experimental import pallas as pl
from jax.experimental.pallas import tpu as pltpu
from jax.experimental.pallas import tpu_sc as plsc

_VOCAB = 1000
_B = 1024
_L = 200
_BL = _B * _L
_NC = 2
_NS = 16
_NW = _NC * _NS
_PER_W = _BL // _NW
_C = 40
_NCHUNK = _PER_W // _C

_mesh = plsc.VectorSubcoreMesh(core_axis_name="c", subcore_axis_name="s")


@functools.partial(
    pl.kernel,
    mesh=_mesh,
    out_type=jax.ShapeDtypeStruct((_BL, _VOCAB), jnp.float32),
    scratch_types=[
        pltpu.VMEM((_PER_W,), jnp.int32),
        pltpu.VMEM((_C, _VOCAB), jnp.float32),
        pltpu.SemaphoreType.DMA,
    ],
    compiler_params=pltpu.CompilerParams(use_tc_tiling_on_sc=False),
)
def _gather_rows(idx_hbm, table_hbm, out_hbm, idx_v, rows_v, sem):
    wid = lax.axis_index("s") * _NC + lax.axis_index("c")
    base = wid * _PER_W
    pltpu.sync_copy(idx_hbm.at[pl.ds(base, _PER_W)], idx_v)

    def body(i, carry):
        off = i * _C
        pltpu.async_copy(
            table_hbm.at[idx_v.at[pl.ds(off, _C)]], rows_v, sem
        ).wait()
        pltpu.sync_copy(rows_v, out_hbm.at[pl.ds(base + off, _C)])
        return carry

    lax.fori_loop(0, _NCHUNK, body, 0)


def kernel(idx, table):
    idx_flat = idx.reshape(-1).astype(jnp.int32)
    out = _gather_rows(idx_flat, table)
    return out.reshape(_B, _L, _VOCAB)


# speedup vs baseline: 1.0328x; 1.0328x over previous
import functools

import jax
import jax.numpy as jnp
from jax import lax
from jax.experimental import pallas as pl
from jax.experimental.pallas import tpu as pltpu
from jax.experimental.pallas import tpu_sc as plsc

_VOCAB = 1000
_B = 1024
_L = 200
_BL = _B * _L
_NC = 2
_NS = 16
_NW = _NC * _NS
_PER_W = _BL // _NW
_C = 40
_NCHUNK = _PER_W // _C
_NBUF = 2

_mesh = plsc.VectorSubcoreMesh(core_axis_name="c", subcore_axis_name="s")


@functools.partial(
    pl.kernel,
    mesh=_mesh,
    out_type=jax.ShapeDtypeStruct((_BL, _VOCAB), jnp.float32),
    scratch_types=[
        pltpu.VMEM((_PER_W,), jnp.int32),
        [pltpu.VMEM((_C, _VOCAB), jnp.float32) for _ in range(_NBUF)],
        [pltpu.SemaphoreType.DMA for _ in range(_NBUF)],
        [pltpu.SemaphoreType.DMA for _ in range(_NBUF)],
    ],
    compiler_params=pltpu.CompilerParams(use_tc_tiling_on_sc=False),
)
def _gather_rows(idx_hbm, table_hbm, out_hbm, idx_v, rows, gsem, wsem):
    wid = lax.axis_index("s") * _NC + lax.axis_index("c")
    base = wid * _PER_W
    pltpu.sync_copy(idx_hbm.at[pl.ds(base, _PER_W)], idx_v)

    def start_gather(g, b):
        pltpu.async_copy(
            table_hbm.at[idx_v.at[pl.ds(g * _C, _C)]], rows[b], gsem[b]
        )

    def wait_gather(g, b):
        pltpu.make_async_copy(
            table_hbm.at[idx_v.at[pl.ds(g * _C, _C)]], rows[b], gsem[b]
        ).wait()

    def start_write(g, b):
        pltpu.async_copy(rows[b], out_hbm.at[pl.ds(base + g * _C, _C)], wsem[b])

    def wait_write(g, b):
        pltpu.make_async_copy(
            rows[b], out_hbm.at[pl.ds(base + g * _C, _C)], wsem[b]
        ).wait()

    for b in range(_NBUF):
        start_gather(b, b)

    def body(step, carry):
        for b in range(_NBUF):
            g = step * _NBUF + b
            wait_gather(g, b)
            start_write(g, b)
        for b in range(_NBUF):
            g = step * _NBUF + b
            nxt = g + _NBUF

            @pl.when(nxt < _NCHUNK)
            def _():
                wait_write(g, b)
                start_gather(nxt, b)

        return carry

    lax.fori_loop(0, _NCHUNK // _NBUF, body, 0)

    for b in range(_NBUF):
        wait_write(_NCHUNK - _NBUF + b, b)


def kernel(idx, table):
    idx_flat = idx.reshape(-1).astype(jnp.int32)
    out = _gather_rows(idx_flat, table)
    return out.reshape(_B, _L, _VOCAB)


# speedup vs baseline: 1.0356x; 1.0027x over previous
import functools

import jax
import jax.numpy as jnp
from jax import lax
from jax.experimental import pallas as pl
from jax.experimental.pallas import tpu as pltpu
from jax.experimental.pallas import tpu_sc as plsc

_VOCAB = 1000
_B = 1024
_L = 200
_BL = _B * _L
_NC = 2
_NS = 16
_NW = _NC * _NS
_PER_W = _BL // _NW
_BPW = _B // _NW
_C = 40
_CPB = _L // _C
_NCHUNK = _PER_W // _C
_NBUF = 2

_mesh = plsc.VectorSubcoreMesh(core_axis_name="c", subcore_axis_name="s")


@functools.partial(
    pl.kernel,
    mesh=_mesh,
    out_type=jax.ShapeDtypeStruct((_B, _L, _VOCAB), jnp.float32),
    scratch_types=[
        pltpu.VMEM((_PER_W,), jnp.int32),
        [pltpu.VMEM((_C, _VOCAB), jnp.float32) for _ in range(_NBUF)],
        [pltpu.SemaphoreType.DMA for _ in range(_NBUF)],
        [pltpu.SemaphoreType.DMA for _ in range(_NBUF)],
    ],
    compiler_params=pltpu.CompilerParams(use_tc_tiling_on_sc=False),
)
def _gather_rows(idx_hbm, table_hbm, out_hbm, idx_v, rows, gsem, wsem):
    wid = lax.axis_index("s") * _NC + lax.axis_index("c")
    base = wid * _PER_W
    pltpu.sync_copy(idx_hbm.at[pl.ds(base, _PER_W)], idx_v)

    def out_slice(g):
        b = wid * _BPW + g // _CPB
        l0 = (g % _CPB) * _C
        return out_hbm.at[b, pl.ds(l0, _C)]

    def start_gather(g, b):
        pltpu.async_copy(
            table_hbm.at[idx_v.at[pl.ds(g * _C, _C)]], rows[b], gsem[b]
        )

    def wait_gather(g, b):
        pltpu.make_async_copy(
            table_hbm.at[idx_v.at[pl.ds(g * _C, _C)]], rows[b], gsem[b]
        ).wait()

    def start_write(g, b):
        pltpu.async_copy(rows[b], out_slice(g), wsem[b])

    def wait_write(g, b):
        pltpu.make_async_copy(rows[b], out_slice(g), wsem[b]).wait()

    for b in range(_NBUF):
        start_gather(b, b)

    def body(step, carry):
        for b in range(_NBUF):
            g = step * _NBUF + b
            wait_gather(g, b)
            start_write(g, b)
        for b in range(_NBUF):
            g = step * _NBUF + b
            nxt = g + _NBUF

            @pl.when(nxt < _NCHUNK)
            def _():
                wait_write(g, b)
                start_gather(nxt, b)

        return carry

    lax.fori_loop(0, _NCHUNK // _NBUF, body, 0)

    for b in range(_NBUF):
        wait_write(_NCHUNK - _NBUF + b, b)


def kernel(idx, table):
    idx_flat = idx.reshape(-1).astype(jnp.int32)
    return _gather_rows(idx_flat, table)


# speedup vs baseline: 1.5526x; 1.4993x over previous
import functools

import jax
import jax.numpy as jnp
from jax import lax
from jax.experimental import pallas as pl
from jax.experimental.pallas import tpu as pltpu
from jax.experimental.pallas import tpu_sc as plsc

_VOCAB = 1000
_VPAD = 1024
_NSTRIP = _VPAD // 128
_TAIL = _VOCAB - 128 * (_NSTRIP - 1)
_B = 1024
_L = 200
_BL = _B * _L
_NC = 2
_NS = 16
_NW = _NC * _NS
_PER_W = _BL // _NW
_BPW = _B // _NW
_C = 40
_CPB = _L // _C
_NCHUNK = _PER_W // _C
_NBUF = 2
_IT = 48

_mesh = plsc.VectorSubcoreMesh(core_axis_name="c", subcore_axis_name="s")


@functools.partial(
    pl.kernel,
    mesh=_mesh,
    out_type=jax.ShapeDtypeStruct((_B, _L, _VOCAB), jnp.float32),
    scratch_types=[
        pltpu.VMEM((_PER_W + 16,), jnp.int32),
        [pltpu.VMEM((_NSTRIP * _IT,), jnp.int32) for _ in range(_NBUF)],
        [pltpu.VMEM((_C, _VOCAB), jnp.float32) for _ in range(_NBUF)],
        [pltpu.VMEM((_C, 128), jnp.float32) for _ in range(_NBUF)],
        [pltpu.SemaphoreType.DMA for _ in range(_NBUF)],
        [pltpu.SemaphoreType.DMA for _ in range(_NBUF)],
    ],
    compiler_params=pltpu.CompilerParams(needs_layout_passes=False),
)
def _gather_rows(idx8_hbm, table_hbm, out_hbm, idx_v, idx_t, rows, tail, gsem, wsem):
    wid = lax.axis_index("s") * _NC + lax.axis_index("c")
    base = wid * _PER_W
    pltpu.sync_copy(idx8_hbm.at[pl.ds(base, _PER_W)], idx_v.at[pl.ds(0, _PER_W)])

    def out_slice(g):
        b = wid * _BPW + g // _CPB
        l0 = (g % _CPB) * _C
        return out_hbm.at[b, pl.ds(l0, _C)]

    def fill_idx(g, b):
        regs = [idx_v[pl.ds(g * _C + 16 * k, 16)] for k in range(3)]
        for t in range(_NSTRIP):
            for k in range(3):
                idx_t[b][pl.ds(t * _IT + 16 * k, 16)] = regs[k] + t

    def strip_dst(b, t):
        if t < _NSTRIP - 1:
            return rows[b].at[:, pl.ds(128 * t, 128)]
        return tail[b]

    def start_gathers(g, b):
        fill_idx(g, b)
        for t in range(_NSTRIP):
            pltpu.async_copy(
                table_hbm.at[idx_t[b].at[pl.ds(t * _IT, _C)]],
                strip_dst(b, t),
                gsem[b],
            )

    def wait_gathers(g, b):
        for t in range(_NSTRIP):
            pltpu.make_async_copy(
                table_hbm.at[idx_t[b].at[pl.ds(t * _IT, _C)]],
                strip_dst(b, t),
                gsem[b],
            ).wait()

    def merge_tail(b):
        lanes = lax.iota(jnp.int32, 16)
        mask = lanes < (_TAIL - 16 * (_TAIL // 16))
        col_ids = 128 * (_NSTRIP - 1) + 16 * (_TAIL // 16) + lanes
        for l in range(_C):
            for k in range(_TAIL // 16):
                rows[b][l, pl.ds(128 * (_NSTRIP - 1) + 16 * k, 16)] = tail[b][
                    l, pl.ds(16 * k, 16)
                ]
            plsc.store_scatter(
                rows[b],
                [jnp.full((16,), l, jnp.int32), col_ids],
                tail[b][l, pl.ds(16 * (_TAIL // 16), 16)],
                mask=mask,
            )

    def start_write(g, b):
        pltpu.async_copy(rows[b], out_slice(g), wsem[b])

    def wait_write(g, b):
        pltpu.make_async_copy(rows[b], out_slice(g), wsem[b]).wait()

    for b in range(_NBUF):
        start_gathers(b, b)

    def body(step, carry):
        for b in range(_NBUF):
            g = step * _NBUF + b
            wait_gathers(g, b)
            merge_tail(b)
            start_write(g, b)
        for b in range(_NBUF):
            g = step * _NBUF + b
            nxt = g + _NBUF

            @pl.when(nxt < _NCHUNK)
            def _():
                wait_write(g, b)
                start_gathers(nxt, b)

        return carry

    lax.fori_loop(0, _NCHUNK // _NBUF, body, 0)

    for b in range(_NBUF):
        wait_write(_NCHUNK - _NBUF + b, b)


def kernel(idx, table):
    idx8 = idx.reshape(-1).astype(jnp.int32) * _NSTRIP
    table_t = jnp.pad(table, ((0, 0), (0, _VPAD - _VOCAB))).reshape(
        _VOCAB * _NSTRIP, 128
    )
    return _gather_rows(idx8, table_t)


# speedup vs baseline: 1.5674x; 1.0095x over previous
import functools

import jax
import jax.numpy as jnp
from jax import lax
from jax.experimental import pallas as pl
from jax.experimental.pallas import tpu as pltpu
from jax.experimental.pallas import tpu_sc as plsc

_VOCAB = 1000
_MAIN = 896
_TAIL = _VOCAB - _MAIN
_B = 1024
_L = 200
_BL = _B * _L
_NC = 2
_NS = 16
_NW = _NC * _NS
_PER_W = _BL // _NW
_BPW = _B // _NW
_C = 40
_CPB = _L // _C
_NCHUNK = _PER_W // _C
_NBUF = 2

_mesh = plsc.VectorSubcoreMesh(core_axis_name="c", subcore_axis_name="s")


@functools.partial(
    pl.kernel,
    mesh=_mesh,
    out_type=jax.ShapeDtypeStruct((_B, _L, _VOCAB), jnp.float32),
    scratch_types=[
        pltpu.VMEM((_PER_W,), jnp.int32),
        [pltpu.VMEM((_C, _MAIN), jnp.float32) for _ in range(_NBUF)],
        [pltpu.VMEM((_C, 128), jnp.float32) for _ in range(_NBUF)],
        [pltpu.VMEM((_C, _TAIL), jnp.float32) for _ in range(_NBUF)],
        [pltpu.SemaphoreType.DMA for _ in range(_NBUF)],
        [pltpu.SemaphoreType.DMA for _ in range(_NBUF)],
    ],
    compiler_params=pltpu.CompilerParams(needs_layout_passes=False),
)
def _gather_rows(
    idx_hbm, main_hbm, tail_hbm, out_hbm, idx_v, rows, t128, t104, gsem, wsem
):
    wid = lax.axis_index("s") * _NC + lax.axis_index("c")
    base = wid * _PER_W
    pltpu.sync_copy(idx_hbm.at[pl.ds(base, _PER_W)], idx_v)

    def out_main(g):
        b = wid * _BPW + g // _CPB
        l0 = (g % _CPB) * _C
        return out_hbm.at[b, pl.ds(l0, _C), pl.ds(0, _MAIN)]

    def out_tail(g):
        b = wid * _BPW + g // _CPB
        l0 = (g % _CPB) * _C
        return out_hbm.at[b, pl.ds(l0, _C), pl.ds(_MAIN, _TAIL)]

    def idx_slice(g):
        return idx_v.at[pl.ds(g * _C, _C)]

    def start_gathers(g, b):
        pltpu.async_copy(main_hbm.at[idx_slice(g)], rows[b], gsem[b])
        pltpu.async_copy(tail_hbm.at[idx_slice(g)], t128[b], gsem[b])

    def wait_gathers(g, b):
        pltpu.make_async_copy(main_hbm.at[idx_slice(g)], rows[b], gsem[b]).wait()
        pltpu.make_async_copy(tail_hbm.at[idx_slice(g)], t128[b], gsem[b]).wait()

    lanes = lax.iota(jnp.int32, 16)
    smask = lanes < (_TAIL - 16 * (_TAIL // 16))
    scol = 16 * (_TAIL // 16) + lanes

    def merge_tail(b):
        for l in range(_C):
            for k in range(_TAIL // 16):
                t104[b][l, pl.ds(16 * k, 16)] = t128[b][l, pl.ds(16 * k, 16)]
            plsc.store_scatter(
                t104[b],
                [jnp.full((16,), l, jnp.int32), scol],
                t128[b][l, pl.ds(16 * (_TAIL // 16), 16)],
                mask=smask,
            )

    def start_writes(g, b):
        pltpu.async_copy(rows[b], out_main(g), wsem[b])
        pltpu.async_copy(t104[b], out_tail(g), wsem[b])

    def wait_writes(g, b):
        pltpu.make_async_copy(rows[b], out_main(g), wsem[b]).wait()
        pltpu.make_async_copy(t104[b], out_tail(g), wsem[b]).wait()

    for b in range(_NBUF):
        start_gathers(b, b)

    def body(step, carry):
        for b in range(_NBUF):
            g = step * _NBUF + b
            wait_gathers(g, b)
            merge_tail(b)
            start_writes(g, b)
        for b in range(_NBUF):
            g = step * _NBUF + b
            nxt = g + _NBUF

            @pl.when(nxt < _NCHUNK)
            def _():
                wait_writes(g, b)
                start_gathers(nxt, b)

        return carry

    lax.fori_loop(0, _NCHUNK // _NBUF, body, 0)

    for b in range(_NBUF):
        wait_writes(_NCHUNK - _NBUF + b, b)


def kernel(idx, table):
    idx_flat = idx.reshape(-1).astype(jnp.int32)
    main_t = table[:, :_MAIN]
    tail_t = jnp.pad(table[:, _MAIN:], ((0, 0), (0, 128 - _TAIL)))
    return _gather_rows(idx_flat, main_t, tail_t)
